# baseline (speedup 1.0000x reference)
import functools

import jax
import jax.numpy as jnp
from jax import lax
from jax.experimental import pallas as pl
from jax.experimental.pallas import tpu as pltpu
from jax.experimental.pallas import tpu_sc as plsc

N = 8192
SC_ROWS = 3072


TC_BR = 512
TC_BC = 8192


def _tc_body(emb_ref, r_ref, val_ref, arg_ref):
    r = pl.program_id(1)

    @pl.when(r == 0)
    def _init():
        val_ref[...] = jnp.zeros_like(val_ref)
        arg_ref[...] = jnp.zeros_like(arg_ref)

    emb = emb_ref[0, :]
    blk = r_ref[...]
    p = blk * emb[:, None]
    m = jnp.max(p, axis=0)
    rows = lax.broadcasted_iota(jnp.int32, p.shape, 0)
    cand = jnp.where(p == m[None, :], rows, N)
    a = jnp.min(cand, axis=0) + (r * TC_BR + SC_ROWS)
    cur = val_ref[0, :]
    upd = m > cur
    val_ref[0, :] = jnp.where(upd, m, cur)
    arg_ref[0, :] = jnp.where(upd, a.astype(jnp.float32), arg_ref[0, :])


def _tc_part(embedding, r_embedding):
    rb0 = SC_ROWS // TC_BR
    grid = (N // TC_BC, (N - SC_ROWS) // TC_BR)
    return pl.pallas_call(
        _tc_body,
        grid=grid,
        in_specs=[
            pl.BlockSpec((1, TC_BR), lambda c, r: (0, r + rb0)),
            pl.BlockSpec((TC_BR, TC_BC), lambda c, r: (r + rb0, c)),
        ],
        out_specs=[
            pl.BlockSpec((1, TC_BC), lambda c, r: (0, c)),
            pl.BlockSpec((1, TC_BC), lambda c, r: (0, c)),
        ],
        out_shape=[
            jax.ShapeDtypeStruct((1, N), jnp.float32),
            jax.ShapeDtypeStruct((1, N), jnp.float32),
        ],
    )(embedding, r_embedding)


def _merge_body(sv_ref, sa_ref, tv_ref, ta_ref, val_ref, arg_ref):
    pv = sv_ref[...]
    pa = sa_ref[...]
    m = jnp.max(pv, axis=0)
    io = lax.broadcasted_iota(jnp.int32, pv.shape, 0)
    k = jnp.min(jnp.where(pv == m[None, :], io, NW), axis=0)
    sa = jnp.sum(pa * (io == k[None, :]).astype(jnp.float32), axis=0)
    tv, ta = tv_ref[0, :], ta_ref[0, :]
    take_sc = m >= tv
    val_ref[0, :] = jnp.where(take_sc, m, tv)
    arg_ref[0, :] = jnp.where(take_sc, sa, ta)


def _merge(sv, sa, tv, ta):
    return pl.pallas_call(
        _merge_body,
        out_shape=[
            jax.ShapeDtypeStruct((1, N), jnp.float32),
            jax.ShapeDtypeStruct((1, N), jnp.float32),
        ],
    )(sv, sa, tv, ta)


NW = 32
SC_TQ = 2048
NQ = N // SC_TQ


def _sc_task(buf, ev, row_base_f, q, ma_v, aa_v):
    def cb_body(cb, _):
        coff = q * SC_TQ + cb * 256
        m = [ma_v[pl.ds(coff + j * 16, 16)] for j in range(16)]
        a = [aa_v[pl.ds(coff + j * 16, 16)] for j in range(16)]
        for k in range(16):
            e = ev[k]
            hf = jnp.full((16,), row_base_f + float(k), jnp.float32)
            for j in range(16):
                rv = buf[k, pl.ds(cb * 256 + j * 16, 16)]
                p = rv * e
                upd = p > m[j]
                m[j] = jnp.where(upd, p, m[j])
                a[j] = jnp.where(upd, hf, a[j])
        for j in range(16):
            ma_v[pl.ds(coff + j * 16, 16)] = m[j]
            aa_v[pl.ds(coff + j * 16, 16)] = a[j]
        return 0

    lax.fori_loop(0, SC_TQ // 256, cb_body, 0)


def _sc_part(emb1d, r_embedding):
    rpw = SC_ROWS // NW
    ntask = (rpw // 16) * NQ
    mesh = plsc.VectorSubcoreMesh(core_axis_name="c", subcore_axis_name="s")

    @functools.partial(
        pl.kernel,
        mesh=mesh,
        out_type=[
            jax.ShapeDtypeStruct((NW, N), jnp.float32),
            jax.ShapeDtypeStruct((NW, N), jnp.float32),
        ],
        scratch_types=[
            pltpu.VMEM((SC_ROWS,), jnp.float32),
            pltpu.VMEM((2, 16, SC_TQ), jnp.float32),
            pltpu.VMEM((N,), jnp.float32),
            pltpu.VMEM((N,), jnp.float32),
            pltpu.SemaphoreType.DMA,
            pltpu.SemaphoreType.DMA,
        ],
    )
    def sc_kernel(emb_hbm, r_hbm, val_hbm, arg_hbm,
                  emb_v, rbuf, ma_v, aa_v, sem0, sem1):
        wid = lax.axis_index("s") * 2 + lax.axis_index("c")
        w0 = wid * rpw
        sems = (sem0, sem1)
        pltpu.sync_copy(emb_hbm.at[pl.ds(0, SC_ROWS)], emb_v)

        zero = jnp.zeros((16,), jnp.float32)

        @pl.loop(0, N // 16)
        def _(i):
            ma_v[pl.ds(i * 16, 16)] = zero
            aa_v[pl.ds(i * 16, 16)] = zero

        qshift = NQ.bit_length() - 1

        def src(t):
            s = lax.shift_right_logical(t, qshift)
            q = lax.bitwise_and(t, NQ - 1)
            return r_hbm.at[pl.ds(w0 + s * 16, 16), pl.ds(q * SC_TQ, SC_TQ)], q

        for b in range(2):
            ref, _ = src(jnp.int32(b))
            pltpu.async_copy(ref, rbuf.at[b], sems[b])

        def pair_body(gp, _):
            for b in range(2):
                t = gp * 2 + b
                s = lax.shift_right_logical(t, qshift)
                q = lax.bitwise_and(t, NQ - 1)
                pltpu.make_async_copy(
                    r_hbm.at[pl.ds(0, 16), pl.ds(0, SC_TQ)],
                    rbuf.at[b], sems[b]).wait()
                ev = emb_v[pl.ds(w0 + s * 16, 16)]
                _sc_task(rbuf.at[b], ev, (w0 + s * 16).astype(jnp.float32),
                         q, ma_v, aa_v)

                @pl.when(t + 2 < ntask)
                def _():
                    ref, _ = src(t + 2)
                    pltpu.async_copy(ref, rbuf.at[b], sems[b])
            return 0

        lax.fori_loop(0, ntask // 2, pair_body, 0)
        pltpu.sync_copy(ma_v, val_hbm.at[wid])
        pltpu.sync_copy(aa_v, arg_hbm.at[wid])

    return sc_kernel(emb1d, r_embedding)


def kernel(embedding, r_embedding):
    sval, sarg = _sc_part(embedding.reshape(N), r_embedding)
    tval, targ = _tc_part(embedding, r_embedding)
    val, arg = _merge(sval, sarg, tval, targ)
    return val, arg[0]
